# baseline (speedup 1.0000x reference)
import functools

import jax
import jax.numpy as jnp
from jax import lax
from jax.experimental import pallas as pl
from jax.experimental.pallas import tpu as pltpu
from jax.experimental.pallas import tpu_sc as plsc

EMBEDDING_DIM = 50
CONTEXT_SIZE = 5
NUM_CORES = 2
NUM_SUBCORES = 16
NUM_WORKERS = NUM_CORES * NUM_SUBCORES
CHUNK = 256


@functools.partial(jax.jit, static_argnames=("n_chunks", "sizes"))
def _sc_gather_sum(emb_f, p_f, s_f, gi, pi, si, n_chunks, sizes):
    v_g, v_p, v_s = sizes
    per_w = n_chunks * CHUNK
    n_cols = NUM_WORKERS * per_w
    dim = EMBEDDING_DIM
    mesh = plsc.VectorSubcoreMesh(core_axis_name="c", subcore_axis_name="s")

    @functools.partial(
        pl.kernel,
        mesh=mesh,
        out_type=jax.ShapeDtypeStruct((dim, n_cols), jnp.float32),
        compiler_params=pltpu.CompilerParams(use_tc_tiling_on_sc=False),
        scratch_types=[
            pltpu.VMEM((n_chunks, CHUNK), jnp.int32),
            pltpu.VMEM((n_chunks, CHUNK), jnp.int32),
            pltpu.VMEM((n_chunks, CHUNK), jnp.int32),
            pltpu.VMEM((dim, CHUNK), jnp.float32),
            pltpu.VMEM((dim, CHUNK), jnp.float32),
            pltpu.VMEM((dim, CHUNK), jnp.float32),
            pltpu.SemaphoreType.DMA,
            pltpu.SemaphoreType.DMA,
            pltpu.SemaphoreType.DMA,
            pltpu.SemaphoreType.DMA,
        ],
    )
    def k(emb_h, p_h, s_h, gi_h, pi_h, si_h, x_h,
          gi_v, pi_v, si_v, gbuf, pbuf, sbuf, sem0, sem1, sem2, sem3):
        wid = lax.axis_index("s") * NUM_CORES + lax.axis_index("c")
        pltpu.sync_copy(gi_h.at[wid], gi_v)
        pltpu.sync_copy(pi_h.at[wid], pi_v)
        pltpu.sync_copy(si_h.at[wid], si_v)
        col0 = wid * per_w

        def body(j, carry):
            handles = []
            for d in range(dim):
                handles.append(pltpu.async_copy(
                    emb_h.at[pl.ds(d * v_g, v_g)].at[gi_v.at[j]],
                    gbuf.at[d], sem0))
                handles.append(pltpu.async_copy(
                    p_h.at[pl.ds(d * v_p, v_p)].at[pi_v.at[j]],
                    pbuf.at[d], sem1))
                handles.append(pltpu.async_copy(
                    s_h.at[pl.ds(d * v_s, v_s)].at[si_v.at[j]],
                    sbuf.at[d], sem2))
            for h in handles:
                h.wait()
            for d in range(dim):
                for t in range(CHUNK // 16):
                    c = pl.ds(t * 16, 16)
                    gbuf[d, c] = gbuf[d, c] + pbuf[d, c] + sbuf[d, c]
            o = pltpu.async_copy(
                gbuf, x_h.at[:, pl.ds(col0 + j * CHUNK, CHUNK)], sem3)
            o.wait()
            return carry

        lax.fori_loop(0, n_chunks, body, 0)

    return k(emb_f, p_f, s_f, gi, pi, si)


def _mlp_body(x_ref, w1_ref, b1_ref, w2_ref, b2_ref, o_ref):
    acc = None
    for c in range(CONTEXT_SIZE):
        part = jnp.dot(w1_ref[c], x_ref[:, c, :],
                       preferred_element_type=jnp.float32)
        acc = part if acc is None else acc + part
    h = jnp.maximum(acc + b1_ref[...], 0.0)
    o = jnp.dot(w2_ref[...], h, preferred_element_type=jnp.float32)
    o = o + b2_ref[...]
    m = jnp.max(o, axis=0, keepdims=True)
    e = jnp.exp(o - m)
    lse = jnp.log(jnp.sum(e, axis=0, keepdims=True))
    o_ref[...] = (o - m) - lse


@jax.jit
def _tc_mlp(X, W1t, b1, W2t, b2):
    dim, ctx, B = X.shape
    blk = 2048
    hidden = W1t.shape[1]
    n_tags = W2t.shape[0]
    grid = (B // blk,)
    return pl.pallas_call(
        _mlp_body,
        grid=grid,
        in_specs=[
            pl.BlockSpec((dim, ctx, blk), lambda i: (0, 0, i)),
            pl.BlockSpec((ctx, hidden, dim), lambda i: (0, 0, 0)),
            pl.BlockSpec((hidden, 1), lambda i: (0, 0)),
            pl.BlockSpec((n_tags, hidden), lambda i: (0, 0)),
            pl.BlockSpec((n_tags, 1), lambda i: (0, 0)),
        ],
        out_specs=pl.BlockSpec((n_tags, blk), lambda i: (0, i)),
        out_shape=jax.ShapeDtypeStruct((n_tags, B), jnp.float32),
    )(X, W1t, b1, W2t, b2)


def kernel(inputs, p_inputs, s_inputs, emb, p_emb, s_emb, W1, b1, W2, b2):
    ctx, batch = inputs.shape
    dim = emb.shape[1]
    n_cols = ctx * batch
    per_w = n_cols // NUM_WORKERS
    n_chunks = per_w // CHUNK

    def prep(ix):
        return ix.reshape(NUM_WORKERS, n_chunks, CHUNK).astype(jnp.int32)

    def flat(tab):
        v = tab.shape[0]
        vpad = -v % 8
        return jnp.pad(tab.T, ((0, 0), (0, vpad))).reshape(-1), v + vpad

    emb_f, v_g = flat(emb)
    p_f, v_p = flat(p_emb)
    s_f, v_s = flat(s_emb)
    X = _sc_gather_sum(emb_f, p_f, s_f,
                       prep(inputs), prep(p_inputs), prep(s_inputs),
                       n_chunks, (v_g, v_p, v_s))
    X = X.reshape(dim, ctx, batch)
    W1t = W1.T.reshape(W1.shape[1], ctx, dim).transpose(1, 0, 2)
    oT = _tc_mlp(X, W1t, b1.reshape(-1, 1), W2.T, b2.reshape(-1, 1))
    return oT.T


# speedup vs baseline: 1.0232x; 1.0232x over previous
import functools

import jax
import jax.numpy as jnp
from jax import lax
from jax.experimental import pallas as pl
from jax.experimental.pallas import tpu as pltpu
from jax.experimental.pallas import tpu_sc as plsc

EMBEDDING_DIM = 50
CONTEXT_SIZE = 5
NUM_CORES = 2
NUM_SUBCORES = 16
NUM_WORKERS = NUM_CORES * NUM_SUBCORES
CHUNK = 256
VEC = 16


@functools.partial(jax.jit, static_argnames=("n_chunks", "sizes"))
def _sc_gather_sum(emb_f, p_f, s_f, gi, pi, si, n_chunks, sizes):
    v_g, v_p, v_s = sizes
    per_w = n_chunks * CHUNK
    n_cols = NUM_WORKERS * per_w
    dim = EMBEDDING_DIM
    mesh = plsc.VectorSubcoreMesh(core_axis_name="c", subcore_axis_name="s")

    @functools.partial(
        pl.kernel,
        mesh=mesh,
        out_type=jax.ShapeDtypeStruct((dim, n_cols), jnp.float32),
        compiler_params=pltpu.CompilerParams(use_tc_tiling_on_sc=False),
        scratch_types=[
            pltpu.VMEM((n_chunks, CHUNK), jnp.int32),
            pltpu.VMEM((n_chunks, CHUNK), jnp.int32),
            pltpu.VMEM((n_chunks, CHUNK), jnp.int32),
            pltpu.VMEM((dim, CHUNK), jnp.int32),
            pltpu.VMEM((dim, CHUNK), jnp.int32),
            pltpu.VMEM((dim, CHUNK), jnp.int32),
            pltpu.VMEM((dim, CHUNK), jnp.float32),
            pltpu.VMEM((dim, CHUNK), jnp.float32),
            pltpu.VMEM((dim, CHUNK), jnp.float32),
            pltpu.SemaphoreType.DMA,
            pltpu.SemaphoreType.DMA,
            pltpu.SemaphoreType.DMA,
            pltpu.SemaphoreType.DMA,
        ],
    )
    def k(emb_h, p_h, s_h, gi_h, pi_h, si_h, x_h,
          gi_v, pi_v, si_v, gix, pix, six, gbuf, pbuf, sbuf,
          sem0, sem1, sem2, sem3):
        wid = lax.axis_index("s") * NUM_CORES + lax.axis_index("c")
        pltpu.sync_copy(gi_h.at[wid], gi_v)
        pltpu.sync_copy(pi_h.at[wid], pi_v)
        pltpu.sync_copy(si_h.at[wid], si_v)
        col0 = wid * per_w

        def body(j, carry):
            def idx_body(d, _):
                for t in range(CHUNK // VEC):
                    c = pl.ds(t * VEC, VEC)
                    gix[d, c] = gi_v[j, c] + d * v_g
                    pix[d, c] = pi_v[j, c] + d * v_p
                    six[d, c] = si_v[j, c] + d * v_s
                return _
            lax.fori_loop(0, dim, idx_body, 0)
            handles = []
            for d in range(dim):
                handles.append(pltpu.async_copy(
                    emb_h.at[gix.at[d]], gbuf.at[d], sem0))
                handles.append(pltpu.async_copy(
                    p_h.at[pix.at[d]], pbuf.at[d], sem1))
                handles.append(pltpu.async_copy(
                    s_h.at[six.at[d]], sbuf.at[d], sem2))
            for h in handles:
                h.wait()

            def sum_body(d, _):
                for t in range(CHUNK // VEC):
                    c = pl.ds(t * VEC, VEC)
                    gbuf[d, c] = gbuf[d, c] + pbuf[d, c] + sbuf[d, c]
                return _
            lax.fori_loop(0, dim, sum_body, 0)
            o = pltpu.async_copy(
                gbuf, x_h.at[:, pl.ds(col0 + j * CHUNK, CHUNK)], sem3)
            o.wait()
            return carry

        lax.fori_loop(0, n_chunks, body, 0)

    return k(emb_f, p_f, s_f, gi, pi, si)


def _mlp_body(x_ref, w1_ref, b1_ref, w2_ref, b2_ref, o_ref):
    acc = None
    for c in range(CONTEXT_SIZE):
        part = jnp.dot(w1_ref[c], x_ref[:, c, :],
                       preferred_element_type=jnp.float32)
        acc = part if acc is None else acc + part
    h = jnp.maximum(acc + b1_ref[...], 0.0)
    o = jnp.dot(w2_ref[...], h, preferred_element_type=jnp.float32)
    o = o + b2_ref[...]
    m = jnp.max(o, axis=0, keepdims=True)
    e = jnp.exp(o - m)
    lse = jnp.log(jnp.sum(e, axis=0, keepdims=True))
    o_ref[...] = (o - m) - lse


@jax.jit
def _tc_mlp(X, W1t, b1, W2t, b2):
    dim, ctx, B = X.shape
    blk = 2048
    hidden = W1t.shape[1]
    n_tags = W2t.shape[0]
    grid = (B // blk,)
    return pl.pallas_call(
        _mlp_body,
        grid=grid,
        in_specs=[
            pl.BlockSpec((dim, ctx, blk), lambda i: (0, 0, i)),
            pl.BlockSpec((ctx, hidden, dim), lambda i: (0, 0, 0)),
            pl.BlockSpec((hidden, 1), lambda i: (0, 0)),
            pl.BlockSpec((n_tags, hidden), lambda i: (0, 0)),
            pl.BlockSpec((n_tags, 1), lambda i: (0, 0)),
        ],
        out_specs=pl.BlockSpec((n_tags, blk), lambda i: (0, i)),
        out_shape=jax.ShapeDtypeStruct((n_tags, B), jnp.float32),
    )(X, W1t, b1, W2t, b2)


def kernel(inputs, p_inputs, s_inputs, emb, p_emb, s_emb, W1, b1, W2, b2):
    ctx, batch = inputs.shape
    dim = emb.shape[1]
    n_cols = ctx * batch
    per_w = n_cols // NUM_WORKERS
    n_chunks = per_w // CHUNK

    def prep(ix):
        return ix.reshape(NUM_WORKERS, n_chunks, CHUNK).astype(jnp.int32)

    def flat(tab):
        return tab.T.reshape(-1), tab.shape[0]

    emb_f, v_g = flat(emb)
    p_f, v_p = flat(p_emb)
    s_f, v_s = flat(s_emb)
    X = _sc_gather_sum(emb_f, p_f, s_f,
                       prep(inputs), prep(p_inputs), prep(s_inputs),
                       n_chunks, (v_g, v_p, v_s))
    X = X.reshape(dim, ctx, batch)
    W1t = W1.T.reshape(W1.shape[1], ctx, dim).transpose(1, 0, 2)
    oT = _tc_mlp(X, W1t, b1.reshape(-1, 1), W2.T, b2.reshape(-1, 1))
    return oT.T
